# baseline (speedup 1.0000x reference)
import functools
import jax
import jax.numpy as jnp
from jax import lax
from jax.experimental import pallas as pl
from jax.experimental.pallas import tpu as pltpu
from jax.experimental.pallas import tpu_sc as plsc

_N = 10000
_E = 320000
_H = 128
_VOCAB = 28
_G = 128

_NC = 2
_NS = 16
_NW = _NC * _NS

_CH = 128
_NBUF = 2
_ECHUNKS = 80
_TCHUNK = _ECHUNKS * _NW
_EP = _TCHUNK * _CH
_QCH = 8
_NPAIR = _QCH // _NBUF
_FC = 0
_CF = 160
_CS = 0
_CS0 = _NS * _CF

_NPF = 10112
_RPF = _NPF // _NS
_NPD = 10240
_RPD = _NPD // _NS


def _deg_body(zeros1_hbm, ones1_hbm, dst_hbm, out_hbm, acc, ones, didx):
    cid = lax.axis_index("c")
    sid = lax.axis_index("s")
    wid = sid * _NC + cid
    pltpu.sync_copy(dst_hbm.at[pl.ds(wid * _ECHUNKS, _ECHUNKS)], didx)
    pltpu.sync_copy(ones1_hbm, ones)
    r0 = sid * _RPD
    pltpu.sync_copy(zeros1_hbm, acc.at[pl.ds(r0, _RPD)])
    plsc.subcore_barrier()

    def body(t, carry):
        pltpu.sync_copy(ones, acc.at[didx.at[t]], add=True)
        return carry

    lax.fori_loop(0, _ECHUNKS, body, 0)
    plsc.subcore_barrier()
    pltpu.sync_copy(acc.at[pl.ds(r0, _RPD)], out_hbm.at[cid, pl.ds(r0, _RPD)])


def _agg_body(g_hbm, src_hbm, dst_hbm, zeros_hbm, out_hbm,
              acc, sidx, didx, rows, sems):
    cid = lax.axis_index("c")
    sid = lax.axis_index("s")
    r0 = sid * _RPF
    pltpu.sync_copy(zeros_hbm, acc.at[pl.ds(r0, _RPF)])
    plsc.subcore_barrier()

    fast = cid == _FC
    nq = lax.select(fast, _CF // _QCH, _CS // _QCH)
    base = lax.select(fast, sid * _CF, _CS0 + sid * _CS)

    def quarter(q, carry):
        pltpu.sync_copy(src_hbm.at[pl.ds(base + q * _QCH, _QCH)], sidx)
        pltpu.sync_copy(dst_hbm.at[pl.ds(base + q * _QCH, _QCH)], didx)

        def body(p, carry2):
            c0 = p * _NBUF
            descs = [
                pltpu.async_copy(g_hbm.at[sidx.at[c0 + j]], rows.at[j],
                                 sems.at[j])
                for j in range(_NBUF)
            ]
            for j in range(_NBUF):
                descs[j].wait()
                pltpu.sync_copy(rows.at[j], acc.at[didx.at[c0 + j]], add=True)
            return carry2

        lax.fori_loop(0, _NPAIR, body, 0)
        return carry

    lax.fori_loop(0, nq, quarter, 0)
    plsc.subcore_barrier()
    pltpu.sync_copy(acc.at[pl.ds(r0, _RPF)], out_hbm.at[cid, pl.ds(r0, _RPF)])


@functools.cache
def _sc_kernels():
    mesh = plsc.VectorSubcoreMesh(
        core_axis_name="c", subcore_axis_name="s",
        num_cores=_NC, num_subcores=_NS)
    deg = pl.kernel(
        _deg_body,
        out_type=jax.ShapeDtypeStruct((_NC, _NPD), jnp.float32),
        mesh=mesh,
        scratch_types=dict(
            acc=pltpu.VMEM_SHARED((_NPD,), jnp.float32),
            ones=pltpu.VMEM((_CH,), jnp.float32),
            didx=pltpu.VMEM((_ECHUNKS, _CH), jnp.int32),
        ),
    )
    agg = pl.kernel(
        _agg_body,
        out_type=jax.ShapeDtypeStruct((_NC, _NPF, _H), jnp.float32),
        mesh=mesh,
        scratch_types=dict(
            acc=pltpu.VMEM_SHARED((_NPF, _H), jnp.float32),
            sidx=pltpu.VMEM((_QCH, _CH), jnp.int32),
            didx=pltpu.VMEM((_QCH, _CH), jnp.int32),
            rows=pltpu.VMEM((_NBUF, _CH, _H), jnp.float32),
            sems=pltpu.SemaphoreType.DMA((_NBUF,)),
        ),
    )
    return deg, agg


def _tc1_body(x_ref, degp_ref, embed_ref, w1_ref, dinv_ref, g1_ref):
    indeg = degp_ref[0, :] + degp_ref[1, :]
    rows = lax.broadcasted_iota(jnp.int32, (_NPF,), 0)
    dinv = jnp.where(rows < _N, lax.rsqrt(indeg + 1.0), 0.0)
    dinv_ref[...] = dinv
    oh = (x_ref[...] == lax.broadcasted_iota(jnp.int32, (_NPF, _VOCAB), 1))
    embw = jnp.dot(embed_ref[...], w1_ref[...],
                   preferred_element_type=jnp.float32)
    hw = jnp.dot(oh.astype(jnp.float32), embw,
                 preferred_element_type=jnp.float32,
                 precision=lax.Precision.HIGHEST)
    g1_ref[...] = dinv[:, None] * hw


def _tc_mid_body(aggp_ref, g_ref, dinv_ref, b_ref, w_ref, gn_ref):
    dinv = dinv_ref[...]
    s = aggp_ref[0] + aggp_ref[1] + g_ref[...]
    h = jax.nn.relu(dinv[:, None] * s + b_ref[...][None, :])
    gn_ref[...] = dinv[:, None] * jnp.dot(h, w_ref[...],
                                          preferred_element_type=jnp.float32)


def _tc4_body(aggp_ref, g_ref, dinv_ref, b_ref, batch_ref,
              mw1_ref, mb1_ref, mw2_ref, mb2_ref, y_ref):
    dinv = dinv_ref[...]
    s = aggp_ref[0] + aggp_ref[1] + g_ref[...]
    h = jax.nn.relu(dinv[:, None] * s + b_ref[...][None, :])
    gid = lax.broadcasted_iota(jnp.int32, (_G, _NPF), 0)
    m = (gid == batch_ref[...][None, :]).astype(jnp.float32)
    y = jnp.dot(m, h, preferred_element_type=jnp.float32,
                precision=lax.Precision.HIGHEST)
    y = jax.nn.relu(jnp.dot(y, mw1_ref[...],
                            preferred_element_type=jnp.float32)
                    + mb1_ref[...][None, :])
    y_ref[...] = (jnp.dot(y, mw2_ref[...], preferred_element_type=jnp.float32)
                  + mb2_ref[...][None, :])


_tc1 = pl.pallas_call(
    _tc1_body,
    out_shape=(jax.ShapeDtypeStruct((_NPF,), jnp.float32),
               jax.ShapeDtypeStruct((_NPF, _H), jnp.float32)),
)

_tc_mid = pl.pallas_call(
    _tc_mid_body,
    out_shape=jax.ShapeDtypeStruct((_NPF, _H), jnp.float32),
)

_tc4 = pl.pallas_call(
    _tc4_body,
    out_shape=jax.ShapeDtypeStruct((_G, 1), jnp.float32),
)


def kernel(x, edge_index, batch, embed, W1, b1, W2, b2, W3, b3,
           mw1, mb1, mw2, mb2):
    src = edge_index[0].astype(jnp.int32)
    dst = edge_index[1].astype(jnp.int32)
    pad = jnp.full((_EP - _E,), _N, dtype=jnp.int32)
    src_p = jnp.concatenate([src, pad]).reshape(_TCHUNK, _CH)
    dst_p = jnp.concatenate([dst, pad]).reshape(_TCHUNK, _CH)
    x_p = jnp.concatenate(
        [x.astype(jnp.int32).reshape(_N, 1),
         jnp.zeros((_NPF - _N, 1), jnp.int32)])
    batch_p = jnp.concatenate(
        [batch.astype(jnp.int32), jnp.full((_NPF - _N,), _G, dtype=jnp.int32)])
    zeros2 = jnp.zeros((_RPF, _H), jnp.float32)
    zeros1 = jnp.zeros((_RPD,), jnp.float32)
    ones1 = jnp.ones((_CH,), jnp.float32)

    _deg_kernel, _agg_kernel = _sc_kernels()
    degp = _deg_kernel(zeros1, ones1, dst_p)
    dinv, g1 = _tc1(x_p, degp[:, :_NPF], embed, W1)
    a1 = _agg_kernel(g1, src_p, dst_p, zeros2)
    g2 = _tc_mid(a1, g1, dinv, b1, W2)
    a2 = _agg_kernel(g2, src_p, dst_p, zeros2)
    g3 = _tc_mid(a2, g2, dinv, b2, W3)
    a3 = _agg_kernel(g3, src_p, dst_p, zeros2)
    y = _tc4(a3, g3, dinv, b3, batch_p, mw1, mb1, mw2, mb2)
    return y[:, 0]


# speedup vs baseline: 1.3902x; 1.3902x over previous
import functools
import jax
import jax.numpy as jnp
from jax import lax
from jax.experimental import pallas as pl
from jax.experimental.pallas import tpu as pltpu
from jax.experimental.pallas import tpu_sc as plsc

_N = 10000
_E = 320000
_H = 128
_VOCAB = 28
_G = 128

_NC = 2
_NS = 16
_NW = _NC * _NS

_CH = 128
_NBUF = 2
_ECHUNKS = 80
_TCHUNK = _ECHUNKS * _NW
_EP = _TCHUNK * _CH
_QCH = 8
_NPAIR = _QCH // _NBUF
_FC = 0
_CF = 120
_CS = 40
_CS0 = _NS * _CF

_NPF = 10112
_RPF = _NPF // _NS
_NPD = 10240
_RPD = _NPD // _NS


def _deg_body(zeros1_hbm, ones1_hbm, dst_hbm, out_hbm, acc, ones, didx):
    cid = lax.axis_index("c")
    sid = lax.axis_index("s")
    wid = sid * _NC + cid
    pltpu.sync_copy(dst_hbm.at[pl.ds(wid * _ECHUNKS, _ECHUNKS)], didx)
    pltpu.sync_copy(ones1_hbm, ones)
    r0 = sid * _RPD
    pltpu.sync_copy(zeros1_hbm, acc.at[pl.ds(r0, _RPD)])
    plsc.subcore_barrier()

    def body(t, carry):
        pltpu.sync_copy(ones, acc.at[didx.at[t]], add=True)
        return carry

    lax.fori_loop(0, _ECHUNKS, body, 0)
    plsc.subcore_barrier()
    pltpu.sync_copy(acc.at[pl.ds(r0, _RPD)], out_hbm.at[cid, pl.ds(r0, _RPD)])


def _agg_body(g_hbm, src_hbm, dst_hbm, zeros_hbm, out_hbm,
              acc, sidx, didx, rows, sems):
    cid = lax.axis_index("c")
    sid = lax.axis_index("s")
    r0 = sid * _RPF
    pltpu.sync_copy(zeros_hbm.at[cid, sid], acc.at[pl.ds(r0, _RPF)])
    plsc.subcore_barrier()

    fast = cid == _FC
    nq = lax.select(fast, _CF // _QCH, _CS // _QCH)
    base = lax.select(fast, sid * _CF, _CS0 + sid * _CS)

    def quarter(q, carry):
        pltpu.sync_copy(src_hbm.at[pl.ds(base + q * _QCH, _QCH)], sidx)
        pltpu.sync_copy(dst_hbm.at[pl.ds(base + q * _QCH, _QCH)], didx)

        def body(p, carry2):
            c0 = p * _NBUF
            descs = [
                pltpu.async_copy(g_hbm.at[sidx.at[c0 + j]], rows.at[j],
                                 sems.at[j])
                for j in range(_NBUF)
            ]
            for j in range(_NBUF):
                descs[j].wait()
                pltpu.sync_copy(rows.at[j], acc.at[didx.at[c0 + j]], add=True)
            return carry2

        lax.fori_loop(0, _NPAIR, body, 0)
        return carry

    lax.fori_loop(0, nq, quarter, 0)
    plsc.subcore_barrier()
    pltpu.sync_copy(acc.at[pl.ds(r0, _RPF)], out_hbm.at[cid, pl.ds(r0, _RPF)])


@functools.cache
def _sc_kernels():
    mesh = plsc.VectorSubcoreMesh(
        core_axis_name="c", subcore_axis_name="s",
        num_cores=_NC, num_subcores=_NS)
    deg = pl.kernel(
        _deg_body,
        out_type=jax.ShapeDtypeStruct((_NC, _NPD), jnp.float32),
        mesh=mesh,
        scratch_types=dict(
            acc=pltpu.VMEM_SHARED((_NPD,), jnp.float32),
            ones=pltpu.VMEM((_CH,), jnp.float32),
            didx=pltpu.VMEM((_ECHUNKS, _CH), jnp.int32),
        ),
    )
    agg = pl.kernel(
        _agg_body,
        out_type=jax.ShapeDtypeStruct((_NC, _NPF, _H), jnp.float32),
        mesh=mesh,
        scratch_types=dict(
            acc=pltpu.VMEM_SHARED((_NPF, _H), jnp.float32),
            sidx=pltpu.VMEM((_QCH, _CH), jnp.int32),
            didx=pltpu.VMEM((_QCH, _CH), jnp.int32),
            rows=pltpu.VMEM((_NBUF, _CH, _H), jnp.float32),
            sems=pltpu.SemaphoreType.DMA((_NBUF,)),
        ),
    )
    return deg, agg


def _tc1_body(x_ref, degp_ref, embed_ref, w1_ref, dinv_ref, g1_ref):
    indeg = degp_ref[0, :] + degp_ref[1, :]
    rows = lax.broadcasted_iota(jnp.int32, (_NPF,), 0)
    dinv = jnp.where(rows < _N, lax.rsqrt(indeg + 1.0), 0.0)
    dinv_ref[...] = dinv
    oh = (x_ref[...] == lax.broadcasted_iota(jnp.int32, (_NPF, _VOCAB), 1))
    embw = jnp.dot(embed_ref[...], w1_ref[...],
                   preferred_element_type=jnp.float32)
    hw = jnp.dot(oh.astype(jnp.float32), embw,
                 preferred_element_type=jnp.float32,
                 precision=lax.Precision.HIGHEST)
    g1_ref[...] = dinv[:, None] * hw


def _tc_mid_body(aggp_ref, g_ref, dinv_ref, b_ref, w_ref, gn_ref):
    dinv = dinv_ref[...]
    s = aggp_ref[0] + aggp_ref[1] + g_ref[...]
    h = jax.nn.relu(dinv[:, None] * s + b_ref[...][None, :])
    gn_ref[...] = dinv[:, None] * jnp.dot(h, w_ref[...],
                                          preferred_element_type=jnp.float32)


def _tc4_body(aggp_ref, g_ref, dinv_ref, b_ref, batch_ref,
              mw1_ref, mb1_ref, mw2_ref, mb2_ref, y_ref):
    dinv = dinv_ref[...]
    s = aggp_ref[0] + aggp_ref[1] + g_ref[...]
    h = jax.nn.relu(dinv[:, None] * s + b_ref[...][None, :])
    gid = lax.broadcasted_iota(jnp.int32, (_G, _NPF), 0)
    m = (gid == batch_ref[...][None, :]).astype(jnp.float32)
    y = jnp.dot(m, h, preferred_element_type=jnp.float32,
                precision=lax.Precision.HIGHEST)
    y = jax.nn.relu(jnp.dot(y, mw1_ref[...],
                            preferred_element_type=jnp.float32)
                    + mb1_ref[...][None, :])
    y_ref[...] = (jnp.dot(y, mw2_ref[...], preferred_element_type=jnp.float32)
                  + mb2_ref[...][None, :])


_tc1 = pl.pallas_call(
    _tc1_body,
    out_shape=(jax.ShapeDtypeStruct((_NPF,), jnp.float32),
               jax.ShapeDtypeStruct((_NPF, _H), jnp.float32)),
)

_tc_mid = pl.pallas_call(
    _tc_mid_body,
    out_shape=jax.ShapeDtypeStruct((_NPF, _H), jnp.float32),
)

_tc4 = pl.pallas_call(
    _tc4_body,
    out_shape=jax.ShapeDtypeStruct((_G, 1), jnp.float32),
)


def kernel(x, edge_index, batch, embed, W1, b1, W2, b2, W3, b3,
           mw1, mb1, mw2, mb2):
    src = edge_index[0].astype(jnp.int32)
    dst = edge_index[1].astype(jnp.int32)
    pad = jnp.full((_EP - _E,), _N, dtype=jnp.int32)
    goff = jnp.where(jnp.arange(_TCHUNK, dtype=jnp.int32) < _CS0,
                     _FC, 1 - _FC) * _NPF
    src_p = (jnp.concatenate([src, pad]).reshape(_TCHUNK, _CH)
             + goff[:, None])
    dst_p = jnp.concatenate([dst, pad]).reshape(_TCHUNK, _CH)
    x_p = jnp.concatenate(
        [x.astype(jnp.int32).reshape(_N, 1),
         jnp.zeros((_NPF - _N, 1), jnp.int32)])
    batch_p = jnp.concatenate(
        [batch.astype(jnp.int32), jnp.full((_NPF - _N,), _G, dtype=jnp.int32)])
    zeros2 = jnp.zeros((_NC, _NS, _RPF, _H), jnp.float32)
    zeros1 = jnp.zeros((_RPD,), jnp.float32)
    ones1 = jnp.ones((_CH,), jnp.float32)

    _deg_kernel, _agg_kernel = _sc_kernels()
    degp = _deg_kernel(zeros1, ones1, dst_p)
    dinv, g1 = _tc1(x_p, degp[:, :_NPF], embed, W1)
    a1 = _agg_kernel(jnp.concatenate([g1, g1]), src_p, dst_p, zeros2)
    g2 = _tc_mid(a1, g1, dinv, b1, W2)
    a2 = _agg_kernel(jnp.concatenate([g2, g2]), src_p, dst_p, zeros2)
    g3 = _tc_mid(a2, g2, dinv, b2, W3)
    a3 = _agg_kernel(jnp.concatenate([g3, g3]), src_p, dst_p, zeros2)
    y = _tc4(a3, g3, dinv, b3, batch_p, mw1, mb1, mw2, mb2)
    return y[:, 0]


# speedup vs baseline: 1.4693x; 1.0569x over previous
import functools
import jax
import jax.numpy as jnp
from jax import lax
from jax.experimental import pallas as pl
from jax.experimental.pallas import tpu as pltpu
from jax.experimental.pallas import tpu_sc as plsc

_N = 10000
_E = 320000
_H = 128
_VOCAB = 28
_G = 128

_NC = 2
_NS = 16
_NW = _NC * _NS

_CH = 128
_NBUF = 2
_ECHUNKS = 80
_TCHUNK = _ECHUNKS * _NW
_EP = _TCHUNK * _CH
_QCH = 8
_NPAIR = _QCH // _NBUF
_FC = 0
_CF = 136
_CS = 24
_CS0 = _NS * _CF

_NPF = 10112
_RPF = _NPF // _NS
_NPD = 10240
_RPD = _NPD // _NS


def _deg_body(zeros1_hbm, ones1_hbm, dst_hbm, out_hbm, acc, ones, didx):
    cid = lax.axis_index("c")
    sid = lax.axis_index("s")
    wid = sid * _NC + cid
    pltpu.sync_copy(dst_hbm.at[pl.ds(wid * _ECHUNKS, _ECHUNKS)], didx)
    pltpu.sync_copy(ones1_hbm, ones)
    r0 = sid * _RPD
    pltpu.sync_copy(zeros1_hbm, acc.at[pl.ds(r0, _RPD)])
    plsc.subcore_barrier()

    def body(t, carry):
        pltpu.sync_copy(ones, acc.at[didx.at[t]], add=True)
        return carry

    lax.fori_loop(0, _ECHUNKS, body, 0)
    plsc.subcore_barrier()
    pltpu.sync_copy(acc.at[pl.ds(r0, _RPD)], out_hbm.at[cid, pl.ds(r0, _RPD)])


def _agg_body(g_hbm, src_hbm, dst_hbm, zeros_hbm, out_hbm,
              acc, sidx, didx, rows, sems):
    cid = lax.axis_index("c")
    sid = lax.axis_index("s")
    r0 = sid * _RPF
    pltpu.sync_copy(zeros_hbm.at[cid, sid], acc.at[pl.ds(r0, _RPF)])
    plsc.subcore_barrier()

    fast = cid == _FC
    nq = lax.select(fast, _CF // _QCH, _CS // _QCH)
    base = lax.select(fast, sid * _CF, _CS0 + sid * _CS)

    def quarter(q, carry):
        pltpu.sync_copy(src_hbm.at[pl.ds(base + q * _QCH, _QCH)], sidx)
        pltpu.sync_copy(dst_hbm.at[pl.ds(base + q * _QCH, _QCH)], didx)

        def body(p, carry2):
            c0 = p * _NBUF
            descs = [
                pltpu.async_copy(g_hbm.at[sidx.at[c0 + j]], rows.at[j],
                                 sems.at[j])
                for j in range(_NBUF)
            ]
            for j in range(_NBUF):
                descs[j].wait()
                pltpu.sync_copy(rows.at[j], acc.at[didx.at[c0 + j]], add=True)
            return carry2

        lax.fori_loop(0, _NPAIR, body, 0)
        return carry

    lax.fori_loop(0, nq, quarter, 0)
    plsc.subcore_barrier()
    pltpu.sync_copy(acc.at[pl.ds(r0, _RPF)], out_hbm.at[cid, pl.ds(r0, _RPF)])


@functools.cache
def _sc_kernels():
    mesh = plsc.VectorSubcoreMesh(
        core_axis_name="c", subcore_axis_name="s",
        num_cores=_NC, num_subcores=_NS)
    deg = pl.kernel(
        _deg_body,
        out_type=jax.ShapeDtypeStruct((_NC, _NPD), jnp.float32),
        mesh=mesh,
        scratch_types=dict(
            acc=pltpu.VMEM_SHARED((_NPD,), jnp.float32),
            ones=pltpu.VMEM((_CH,), jnp.float32),
            didx=pltpu.VMEM((_ECHUNKS, _CH), jnp.int32),
        ),
    )
    agg = pl.kernel(
        _agg_body,
        out_type=jax.ShapeDtypeStruct((_NC, _NPF, _H), jnp.float32),
        mesh=mesh,
        scratch_types=dict(
            acc=pltpu.VMEM_SHARED((_NPF, _H), jnp.float32),
            sidx=pltpu.VMEM((_QCH, _CH), jnp.int32),
            didx=pltpu.VMEM((_QCH, _CH), jnp.int32),
            rows=pltpu.VMEM((_NBUF, _CH, _H), jnp.float32),
            sems=pltpu.SemaphoreType.DMA((_NBUF,)),
        ),
    )
    return deg, agg


def _tc1_body(x_ref, degp_ref, embed_ref, w1_ref, dinv_ref, g1_ref):
    indeg = degp_ref[0, :] + degp_ref[1, :]
    rows = lax.broadcasted_iota(jnp.int32, (_NPF,), 0)
    dinv = jnp.where(rows < _N, lax.rsqrt(indeg + 1.0), 0.0)
    dinv_ref[...] = dinv
    oh = (x_ref[...] == lax.broadcasted_iota(jnp.int32, (_NPF, _VOCAB), 1))
    embw = jnp.dot(embed_ref[...], w1_ref[...],
                   preferred_element_type=jnp.float32)
    hw = jnp.dot(oh.astype(jnp.float32), embw,
                 preferred_element_type=jnp.float32,
                 precision=lax.Precision.HIGHEST)
    g1_ref[...] = dinv[:, None] * hw


def _tc_mid_body(aggp_ref, g_ref, dinv_ref, b_ref, w_ref, gn_ref):
    dinv = dinv_ref[...]
    s = aggp_ref[0] + aggp_ref[1] + g_ref[...]
    h = jax.nn.relu(dinv[:, None] * s + b_ref[...][None, :])
    gn_ref[...] = dinv[:, None] * jnp.dot(h, w_ref[...],
                                          preferred_element_type=jnp.float32)


def _tc4_body(aggp_ref, g_ref, dinv_ref, b_ref, batch_ref,
              mw1_ref, mb1_ref, mw2_ref, mb2_ref, y_ref):
    dinv = dinv_ref[...]
    s = aggp_ref[0] + aggp_ref[1] + g_ref[...]
    h = jax.nn.relu(dinv[:, None] * s + b_ref[...][None, :])
    gid = lax.broadcasted_iota(jnp.int32, (_G, _NPF), 0)
    m = (gid == batch_ref[...][None, :]).astype(jnp.float32)
    y = jnp.dot(m, h, preferred_element_type=jnp.float32,
                precision=lax.Precision.HIGHEST)
    y = jax.nn.relu(jnp.dot(y, mw1_ref[...],
                            preferred_element_type=jnp.float32)
                    + mb1_ref[...][None, :])
    y_ref[...] = (jnp.dot(y, mw2_ref[...], preferred_element_type=jnp.float32)
                  + mb2_ref[...][None, :])


_tc1 = pl.pallas_call(
    _tc1_body,
    out_shape=(jax.ShapeDtypeStruct((_NPF,), jnp.float32),
               jax.ShapeDtypeStruct((_NPF, _H), jnp.float32)),
)

_tc_mid = pl.pallas_call(
    _tc_mid_body,
    out_shape=jax.ShapeDtypeStruct((_NPF, _H), jnp.float32),
)

_tc4 = pl.pallas_call(
    _tc4_body,
    out_shape=jax.ShapeDtypeStruct((_G, 1), jnp.float32),
)


def kernel(x, edge_index, batch, embed, W1, b1, W2, b2, W3, b3,
           mw1, mb1, mw2, mb2):
    src = edge_index[0].astype(jnp.int32)
    dst = edge_index[1].astype(jnp.int32)
    pad = jnp.full((_EP - _E,), _N, dtype=jnp.int32)
    goff = jnp.where(jnp.arange(_TCHUNK, dtype=jnp.int32) < _CS0,
                     _FC, 1 - _FC) * _NPF
    src_p = (jnp.concatenate([src, pad]).reshape(_TCHUNK, _CH)
             + goff[:, None])
    dst_p = jnp.concatenate([dst, pad]).reshape(_TCHUNK, _CH)
    x_p = jnp.concatenate(
        [x.astype(jnp.int32).reshape(_N, 1),
         jnp.zeros((_NPF - _N, 1), jnp.int32)])
    batch_p = jnp.concatenate(
        [batch.astype(jnp.int32), jnp.full((_NPF - _N,), _G, dtype=jnp.int32)])
    zeros2 = jnp.zeros((_NC, _NS, _RPF, _H), jnp.float32)
    zeros1 = jnp.zeros((_RPD,), jnp.float32)
    ones1 = jnp.ones((_CH,), jnp.float32)

    _deg_kernel, _agg_kernel = _sc_kernels()
    degp = _deg_kernel(zeros1, ones1, dst_p)
    dinv, g1 = _tc1(x_p, degp[:, :_NPF], embed, W1)
    a1 = _agg_kernel(jnp.concatenate([g1, g1]), src_p, dst_p, zeros2)
    g2 = _tc_mid(a1, g1, dinv, b1, W2)
    a2 = _agg_kernel(jnp.concatenate([g2, g2]), src_p, dst_p, zeros2)
    g3 = _tc_mid(a2, g2, dinv, b2, W3)
    a3 = _agg_kernel(jnp.concatenate([g3, g3]), src_p, dst_p, zeros2)
    y = _tc4(a3, g3, dinv, b3, batch_p, mw1, mb1, mw2, mb2)
    return y[:, 0]


# speedup vs baseline: 1.5567x; 1.0595x over previous
import functools
import jax
import jax.numpy as jnp
from jax import lax
from jax.experimental import pallas as pl
from jax.experimental.pallas import tpu as pltpu
from jax.experimental.pallas import tpu_sc as plsc

_N = 10000
_E = 320000
_H = 128
_VOCAB = 28
_G = 128

_NC = 2
_NS = 16
_NW = _NC * _NS

_CH = 128
_NBUF = 2
_ECHUNKS = 80
_TCHUNK = _ECHUNKS * _NW
_EP = _TCHUNK * _CH
_QCH = 8
_NPAIR = _QCH // _NBUF
_FC = 0
_CF = 144
_CS = 16
_CS0 = _NS * _CF

_NPF = 10112
_RPF = _NPF // _NS
_NPD = 10240
_RPD = _NPD // _NS


def _deg_body(zeros1_hbm, ones1_hbm, dst_hbm, out_hbm, acc, ones, didx):
    cid = lax.axis_index("c")
    sid = lax.axis_index("s")
    wid = sid * _NC + cid
    pltpu.sync_copy(dst_hbm.at[pl.ds(wid * _ECHUNKS, _ECHUNKS)], didx)
    pltpu.sync_copy(ones1_hbm, ones)
    r0 = sid * _RPD
    pltpu.sync_copy(zeros1_hbm, acc.at[pl.ds(r0, _RPD)])
    plsc.subcore_barrier()

    def body(t, carry):
        pltpu.sync_copy(ones, acc.at[didx.at[t]], add=True)
        return carry

    lax.fori_loop(0, _ECHUNKS, body, 0)
    plsc.subcore_barrier()
    pltpu.sync_copy(acc.at[pl.ds(r0, _RPD)], out_hbm.at[cid, pl.ds(r0, _RPD)])


def _agg_body(g_hbm, src_hbm, dst_hbm, zeros_hbm, out_hbm,
              acc, sidx, didx, rows, sems):
    cid = lax.axis_index("c")
    sid = lax.axis_index("s")
    r0 = sid * _RPF
    pltpu.sync_copy(zeros_hbm.at[cid, sid], acc.at[pl.ds(r0, _RPF)])
    plsc.subcore_barrier()

    fast = cid == _FC
    nq = lax.select(fast, _CF // _QCH, _CS // _QCH)
    base = lax.select(fast, sid * _CF, _CS0 + sid * _CS)

    def quarter(q, carry):
        pltpu.sync_copy(src_hbm.at[pl.ds(base + q * _QCH, _QCH)], sidx)
        pltpu.sync_copy(dst_hbm.at[pl.ds(base + q * _QCH, _QCH)], didx)

        def body(p, carry2):
            c0 = p * _NBUF
            descs = [
                pltpu.async_copy(g_hbm.at[sidx.at[c0 + j]], rows.at[j],
                                 sems.at[j])
                for j in range(_NBUF)
            ]
            for j in range(_NBUF):
                descs[j].wait()
                pltpu.sync_copy(rows.at[j], acc.at[didx.at[c0 + j]], add=True)
            return carry2

        lax.fori_loop(0, _NPAIR, body, 0)
        return carry

    lax.fori_loop(0, nq, quarter, 0)
    plsc.subcore_barrier()
    pltpu.sync_copy(acc.at[pl.ds(r0, _RPF)], out_hbm.at[cid, pl.ds(r0, _RPF)])


@functools.cache
def _sc_kernels():
    mesh = plsc.VectorSubcoreMesh(
        core_axis_name="c", subcore_axis_name="s",
        num_cores=_NC, num_subcores=_NS)
    deg = pl.kernel(
        _deg_body,
        out_type=jax.ShapeDtypeStruct((_NC, _NPD), jnp.float32),
        mesh=mesh,
        scratch_types=dict(
            acc=pltpu.VMEM_SHARED((_NPD,), jnp.float32),
            ones=pltpu.VMEM((_CH,), jnp.float32),
            didx=pltpu.VMEM((_ECHUNKS, _CH), jnp.int32),
        ),
    )
    agg = pl.kernel(
        _agg_body,
        out_type=jax.ShapeDtypeStruct((_NC, _NPF, _H), jnp.float32),
        mesh=mesh,
        scratch_types=dict(
            acc=pltpu.VMEM_SHARED((_NPF, _H), jnp.float32),
            sidx=pltpu.VMEM((_QCH, _CH), jnp.int32),
            didx=pltpu.VMEM((_QCH, _CH), jnp.int32),
            rows=pltpu.VMEM((_NBUF, _CH, _H), jnp.float32),
            sems=pltpu.SemaphoreType.DMA((_NBUF,)),
        ),
    )
    return deg, agg


def _tc1_body(x_ref, degp_ref, embed_ref, w1_ref, dinv_ref, g1_ref):
    indeg = degp_ref[0, :] + degp_ref[1, :]
    rows = lax.broadcasted_iota(jnp.int32, (_NPF,), 0)
    dinv = jnp.where(rows < _N, lax.rsqrt(indeg + 1.0), 0.0)
    dinv_ref[...] = dinv
    oh = (x_ref[...] == lax.broadcasted_iota(jnp.int32, (_NPF, _VOCAB), 1))
    embw = jnp.dot(embed_ref[...], w1_ref[...],
                   preferred_element_type=jnp.float32)
    hw = jnp.dot(oh.astype(jnp.float32), embw,
                 preferred_element_type=jnp.float32,
                 precision=lax.Precision.HIGHEST)
    g1_ref[...] = dinv[:, None] * hw


def _tc_mid_body(aggp_ref, g_ref, dinv_ref, b_ref, w_ref, gn_ref):
    dinv = dinv_ref[...]
    s = aggp_ref[0] + aggp_ref[1] + g_ref[...]
    h = jax.nn.relu(dinv[:, None] * s + b_ref[...][None, :])
    gn_ref[...] = dinv[:, None] * jnp.dot(h, w_ref[...],
                                          preferred_element_type=jnp.float32)


def _tc4_body(aggp_ref, g_ref, dinv_ref, b_ref, batch_ref,
              mw1_ref, mb1_ref, mw2_ref, mb2_ref, y_ref):
    dinv = dinv_ref[...]
    s = aggp_ref[0] + aggp_ref[1] + g_ref[...]
    h = jax.nn.relu(dinv[:, None] * s + b_ref[...][None, :])
    gid = lax.broadcasted_iota(jnp.int32, (_G, _NPF), 0)
    m = (gid == batch_ref[...][None, :]).astype(jnp.float32)
    y = jnp.dot(m, h, preferred_element_type=jnp.float32,
                precision=lax.Precision.HIGHEST)
    y = jax.nn.relu(jnp.dot(y, mw1_ref[...],
                            preferred_element_type=jnp.float32)
                    + mb1_ref[...][None, :])
    y_ref[...] = (jnp.dot(y, mw2_ref[...], preferred_element_type=jnp.float32)
                  + mb2_ref[...][None, :])


_tc1 = pl.pallas_call(
    _tc1_body,
    out_shape=(jax.ShapeDtypeStruct((_NPF,), jnp.float32),
               jax.ShapeDtypeStruct((_NPF, _H), jnp.float32)),
)

_tc_mid = pl.pallas_call(
    _tc_mid_body,
    out_shape=jax.ShapeDtypeStruct((_NPF, _H), jnp.float32),
)

_tc4 = pl.pallas_call(
    _tc4_body,
    out_shape=jax.ShapeDtypeStruct((_G, 1), jnp.float32),
)


def kernel(x, edge_index, batch, embed, W1, b1, W2, b2, W3, b3,
           mw1, mb1, mw2, mb2):
    src = edge_index[0].astype(jnp.int32)
    dst = edge_index[1].astype(jnp.int32)
    pad = jnp.full((_EP - _E,), _N, dtype=jnp.int32)
    goff = jnp.where(jnp.arange(_TCHUNK, dtype=jnp.int32) < _CS0,
                     _FC, 1 - _FC) * _NPF
    src_p = (jnp.concatenate([src, pad]).reshape(_TCHUNK, _CH)
             + goff[:, None])
    dst_p = jnp.concatenate([dst, pad]).reshape(_TCHUNK, _CH)
    x_p = jnp.concatenate(
        [x.astype(jnp.int32).reshape(_N, 1),
         jnp.zeros((_NPF - _N, 1), jnp.int32)])
    batch_p = jnp.concatenate(
        [batch.astype(jnp.int32), jnp.full((_NPF - _N,), _G, dtype=jnp.int32)])
    zeros2 = jnp.zeros((_NC, _NS, _RPF, _H), jnp.float32)
    zeros1 = jnp.zeros((_RPD,), jnp.float32)
    ones1 = jnp.ones((_CH,), jnp.float32)

    _deg_kernel, _agg_kernel = _sc_kernels()
    degp = _deg_kernel(zeros1, ones1, dst_p)
    dinv, g1 = _tc1(x_p, degp[:, :_NPF], embed, W1)
    a1 = _agg_kernel(jnp.concatenate([g1, g1]), src_p, dst_p, zeros2)
    g2 = _tc_mid(a1, g1, dinv, b1, W2)
    a2 = _agg_kernel(jnp.concatenate([g2, g2]), src_p, dst_p, zeros2)
    g3 = _tc_mid(a2, g2, dinv, b2, W3)
    a3 = _agg_kernel(jnp.concatenate([g3, g3]), src_p, dst_p, zeros2)
    y = _tc4(a3, g3, dinv, b3, batch_p, mw1, mb1, mw2, mb2)
    return y[:, 0]


# speedup vs baseline: 1.5676x; 1.0070x over previous
import functools
import jax
import jax.numpy as jnp
from jax import lax
from jax.experimental import pallas as pl
from jax.experimental.pallas import tpu as pltpu
from jax.experimental.pallas import tpu_sc as plsc

_N = 10000
_E = 320000
_H = 128
_VOCAB = 28
_G = 128

_NC = 2
_NS = 16
_NW = _NC * _NS

_CH = 128
_NBUF = 2
_ECHUNKS = 80
_TCHUNK = _ECHUNKS * _NW
_EP = _TCHUNK * _CH
_QCH = 8
_NPAIR = _QCH // _NBUF
_FC = 0
_CF = 152
_CS = 8
_CS0 = _NS * _CF

_NPF = 10112
_RPF = _NPF // _NS
_NPD = 10240
_RPD = _NPD // _NS


def _deg_body(zeros1_hbm, ones1_hbm, dst_hbm, out_hbm, acc, ones, didx):
    cid = lax.axis_index("c")
    sid = lax.axis_index("s")
    wid = sid * _NC + cid
    pltpu.sync_copy(dst_hbm.at[pl.ds(wid * _ECHUNKS, _ECHUNKS)], didx)
    pltpu.sync_copy(ones1_hbm, ones)
    r0 = sid * _RPD
    pltpu.sync_copy(zeros1_hbm, acc.at[pl.ds(r0, _RPD)])
    plsc.subcore_barrier()

    def body(t, carry):
        pltpu.sync_copy(ones, acc.at[didx.at[t]], add=True)
        return carry

    lax.fori_loop(0, _ECHUNKS, body, 0)
    plsc.subcore_barrier()
    pltpu.sync_copy(acc.at[pl.ds(r0, _RPD)], out_hbm.at[cid, pl.ds(r0, _RPD)])


def _agg_body(g_hbm, src_hbm, dst_hbm, zeros_hbm, out_hbm,
              acc, sidx, didx, rows, sems):
    cid = lax.axis_index("c")
    sid = lax.axis_index("s")
    r0 = sid * _RPF
    pltpu.sync_copy(zeros_hbm.at[cid, sid], acc.at[pl.ds(r0, _RPF)])
    plsc.subcore_barrier()

    fast = cid == _FC
    nq = lax.select(fast, _CF // _QCH, _CS // _QCH)
    base = lax.select(fast, sid * _CF, _CS0 + sid * _CS)

    def quarter(q, carry):
        pltpu.sync_copy(src_hbm.at[pl.ds(base + q * _QCH, _QCH)], sidx)
        pltpu.sync_copy(dst_hbm.at[pl.ds(base + q * _QCH, _QCH)], didx)

        def body(p, carry2):
            c0 = p * _NBUF
            descs = [
                pltpu.async_copy(g_hbm.at[sidx.at[c0 + j]], rows.at[j],
                                 sems.at[j])
                for j in range(_NBUF)
            ]
            for j in range(_NBUF):
                descs[j].wait()
                pltpu.sync_copy(rows.at[j], acc.at[didx.at[c0 + j]], add=True)
            return carry2

        lax.fori_loop(0, _NPAIR, body, 0)
        return carry

    lax.fori_loop(0, nq, quarter, 0)
    plsc.subcore_barrier()
    pltpu.sync_copy(acc.at[pl.ds(r0, _RPF)], out_hbm.at[cid, pl.ds(r0, _RPF)])


@functools.cache
def _sc_kernels():
    mesh = plsc.VectorSubcoreMesh(
        core_axis_name="c", subcore_axis_name="s",
        num_cores=_NC, num_subcores=_NS)
    deg = pl.kernel(
        _deg_body,
        out_type=jax.ShapeDtypeStruct((_NC, _NPD), jnp.float32),
        mesh=mesh,
        scratch_types=dict(
            acc=pltpu.VMEM_SHARED((_NPD,), jnp.float32),
            ones=pltpu.VMEM((_CH,), jnp.float32),
            didx=pltpu.VMEM((_ECHUNKS, _CH), jnp.int32),
        ),
    )
    agg = pl.kernel(
        _agg_body,
        out_type=jax.ShapeDtypeStruct((_NC, _NPF, _H), jnp.float32),
        mesh=mesh,
        scratch_types=dict(
            acc=pltpu.VMEM_SHARED((_NPF, _H), jnp.float32),
            sidx=pltpu.VMEM((_QCH, _CH), jnp.int32),
            didx=pltpu.VMEM((_QCH, _CH), jnp.int32),
            rows=pltpu.VMEM((_NBUF, _CH, _H), jnp.float32),
            sems=pltpu.SemaphoreType.DMA((_NBUF,)),
        ),
    )
    return deg, agg


def _tc1_body(x_ref, degp_ref, embed_ref, w1_ref, dinv_ref, g1_ref):
    indeg = degp_ref[0, :] + degp_ref[1, :]
    rows = lax.broadcasted_iota(jnp.int32, (_NPF,), 0)
    dinv = jnp.where(rows < _N, lax.rsqrt(indeg + 1.0), 0.0)
    dinv_ref[...] = dinv
    oh = (x_ref[...] == lax.broadcasted_iota(jnp.int32, (_NPF, _VOCAB), 1))
    embw = jnp.dot(embed_ref[...], w1_ref[...],
                   preferred_element_type=jnp.float32)
    hw = jnp.dot(oh.astype(jnp.float32), embw,
                 preferred_element_type=jnp.float32,
                 precision=lax.Precision.HIGHEST)
    g1_ref[...] = dinv[:, None] * hw


def _tc_mid_body(aggp_ref, g_ref, dinv_ref, b_ref, w_ref, gn_ref):
    dinv = dinv_ref[...]
    s = aggp_ref[0] + aggp_ref[1] + g_ref[...]
    h = jax.nn.relu(dinv[:, None] * s + b_ref[...][None, :])
    gn_ref[...] = dinv[:, None] * jnp.dot(h, w_ref[...],
                                          preferred_element_type=jnp.float32)


def _tc4_body(aggp_ref, g_ref, dinv_ref, b_ref, batch_ref,
              mw1_ref, mb1_ref, mw2_ref, mb2_ref, y_ref):
    dinv = dinv_ref[...]
    s = aggp_ref[0] + aggp_ref[1] + g_ref[...]
    h = jax.nn.relu(dinv[:, None] * s + b_ref[...][None, :])
    gid = lax.broadcasted_iota(jnp.int32, (_G, _NPF), 0)
    m = (gid == batch_ref[...][None, :]).astype(jnp.float32)
    y = jnp.dot(m, h, preferred_element_type=jnp.float32,
                precision=lax.Precision.HIGHEST)
    y = jax.nn.relu(jnp.dot(y, mw1_ref[...],
                            preferred_element_type=jnp.float32)
                    + mb1_ref[...][None, :])
    y_ref[...] = (jnp.dot(y, mw2_ref[...], preferred_element_type=jnp.float32)
                  + mb2_ref[...][None, :])


_tc1 = pl.pallas_call(
    _tc1_body,
    out_shape=(jax.ShapeDtypeStruct((_NPF,), jnp.float32),
               jax.ShapeDtypeStruct((_NPF, _H), jnp.float32)),
)

_tc_mid = pl.pallas_call(
    _tc_mid_body,
    out_shape=jax.ShapeDtypeStruct((_NPF, _H), jnp.float32),
)

_tc4 = pl.pallas_call(
    _tc4_body,
    out_shape=jax.ShapeDtypeStruct((_G, 1), jnp.float32),
)


def kernel(x, edge_index, batch, embed, W1, b1, W2, b2, W3, b3,
           mw1, mb1, mw2, mb2):
    src = edge_index[0].astype(jnp.int32)
    dst = edge_index[1].astype(jnp.int32)
    pad = jnp.full((_EP - _E,), _N, dtype=jnp.int32)
    goff = jnp.where(jnp.arange(_TCHUNK, dtype=jnp.int32) < _CS0,
                     _FC, 1 - _FC) * _NPF
    src_p = (jnp.concatenate([src, pad]).reshape(_TCHUNK, _CH)
             + goff[:, None])
    dst_p = jnp.concatenate([dst, pad]).reshape(_TCHUNK, _CH)
    x_p = jnp.concatenate(
        [x.astype(jnp.int32).reshape(_N, 1),
         jnp.zeros((_NPF - _N, 1), jnp.int32)])
    batch_p = jnp.concatenate(
        [batch.astype(jnp.int32), jnp.full((_NPF - _N,), _G, dtype=jnp.int32)])
    zeros2 = jnp.zeros((_NC, _NS, _RPF, _H), jnp.float32)
    zeros1 = jnp.zeros((_RPD,), jnp.float32)
    ones1 = jnp.ones((_CH,), jnp.float32)

    _deg_kernel, _agg_kernel = _sc_kernels()
    degp = _deg_kernel(zeros1, ones1, dst_p)
    dinv, g1 = _tc1(x_p, degp[:, :_NPF], embed, W1)
    a1 = _agg_kernel(jnp.concatenate([g1, g1]), src_p, dst_p, zeros2)
    g2 = _tc_mid(a1, g1, dinv, b1, W2)
    a2 = _agg_kernel(jnp.concatenate([g2, g2]), src_p, dst_p, zeros2)
    g3 = _tc_mid(a2, g2, dinv, b2, W3)
    a3 = _agg_kernel(jnp.concatenate([g3, g3]), src_p, dst_p, zeros2)
    y = _tc4(a3, g3, dinv, b3, batch_p, mw1, mb1, mw2, mb2)
    return y[:, 0]
